# baseline (speedup 1.0000x reference)
import jax
import jax.numpy as jnp
from jax.experimental import pallas as pl
from jax.experimental.pallas import tpu as pltpu

_BH_B = 128
_S_B = 128


def _make_body(Q, D):
    def body(s_ref, cache_ref, k_ref, out_ref):
        b = pl.program_id(1)
        start = s_ref[0]
        b0 = start // _S_B
        o0 = start - b0 * _S_B
        hit = jnp.logical_or(b == b0, b == b0 + 1)

        @pl.when(hit)
        def _():
            kb = k_ref[...]
            kpad = jnp.concatenate(
                [kb, jnp.zeros((_BH_B, _S_B - Q, D), kb.dtype)], axis=1)
            rolled = pltpu.roll(kpad, o0, axis=1)
            r = jax.lax.broadcasted_iota(jnp.int32, kpad.shape, 1)
            mask = jnp.logical_xor(r >= o0, b != b0)
            out_ref[...] = cache_ref[...] + jnp.where(mask, rolled, 0.0)

        @pl.when(jnp.logical_not(hit))
        def _():
            out_ref[...] = cache_ref[...]

    return body


def kernel(k_cache, k, step):
    B, H, S, D = k_cache.shape
    Q = k.shape[-2]
    BH = B * H
    start = jnp.clip(jnp.asarray(step, jnp.int32) - Q, 0, S - Q)

    kc = k_cache.reshape(BH, S, D)
    kk = k.reshape(BH, Q, D)

    cache_spec = pl.BlockSpec(
        (_BH_B, _S_B, D), lambda i, j, s_ref: (i, j, 0))
    k_spec = pl.BlockSpec((_BH_B, Q, D), lambda i, j, s_ref: (i, 0, 0))

    grid_spec = pltpu.PrefetchScalarGridSpec(
        num_scalar_prefetch=1,
        grid=(BH // _BH_B, S // _S_B),
        in_specs=[cache_spec, k_spec],
        out_specs=cache_spec,
    )
    out = pl.pallas_call(
        _make_body(Q, D),
        grid_spec=grid_spec,
        out_shape=jax.ShapeDtypeStruct(kc.shape, kc.dtype),
        compiler_params=pltpu.CompilerParams(
            dimension_semantics=("parallel", "parallel"),
        ),
    )(start.reshape(1), kc, kk)
    return out.reshape(B, H, S, D)


# speedup vs baseline: 1.0160x; 1.0160x over previous
import jax
import jax.numpy as jnp
from jax.experimental import pallas as pl
from jax.experimental.pallas import tpu as pltpu

_BH_B = 64
_S_B = 256


def _make_body(Q, D):
    def body(s_ref, cache_ref, k_ref, out_ref):
        b = pl.program_id(1)
        start = s_ref[0]
        b0 = start // _S_B
        o0 = start - b0 * _S_B
        hit = jnp.logical_or(b == b0, b == b0 + 1)

        @pl.when(hit)
        def _():
            kb = k_ref[...]
            kpad = jnp.concatenate(
                [kb, jnp.zeros((_BH_B, _S_B - Q, D), kb.dtype)], axis=1)
            rolled = pltpu.roll(kpad, o0, axis=1)
            r = jax.lax.broadcasted_iota(jnp.int32, kpad.shape, 1)
            mask = jnp.logical_xor(r >= o0, b != b0)
            out_ref[...] = cache_ref[...] + jnp.where(mask, rolled, 0.0)

        @pl.when(jnp.logical_not(hit))
        def _():
            out_ref[...] = cache_ref[...]

    return body


def kernel(k_cache, k, step):
    B, H, S, D = k_cache.shape
    Q = k.shape[-2]
    BH = B * H
    start = jnp.clip(jnp.asarray(step, jnp.int32) - Q, 0, S - Q)

    kc = k_cache.reshape(BH, S, D)
    kk = k.reshape(BH, Q, D)

    cache_spec = pl.BlockSpec(
        (_BH_B, _S_B, D), lambda i, j, s_ref: (i, j, 0))
    k_spec = pl.BlockSpec((_BH_B, Q, D), lambda i, j, s_ref: (i, 0, 0))

    grid_spec = pltpu.PrefetchScalarGridSpec(
        num_scalar_prefetch=1,
        grid=(BH // _BH_B, S // _S_B),
        in_specs=[cache_spec, k_spec],
        out_specs=cache_spec,
    )
    out = pl.pallas_call(
        _make_body(Q, D),
        grid_spec=grid_spec,
        out_shape=jax.ShapeDtypeStruct(kc.shape, kc.dtype),
        compiler_params=pltpu.CompilerParams(
            dimension_semantics=("parallel", "parallel"),
        ),
    )(start.reshape(1), kc, kk)
    return out.reshape(B, H, S, D)


# speedup vs baseline: 1.0243x; 1.0081x over previous
import jax
import jax.numpy as jnp
from jax.experimental import pallas as pl
from jax.experimental.pallas import tpu as pltpu

_BH_B = 16
_S_B = 1024


def _make_body(Q, D):
    def body(s_ref, cache_ref, k_ref, out_ref):
        b = pl.program_id(1)
        start = s_ref[0]
        b0 = start // _S_B
        o0 = start - b0 * _S_B
        hit = jnp.logical_or(b == b0, b == b0 + 1)

        @pl.when(hit)
        def _():
            kb = k_ref[...]
            kpad = jnp.concatenate(
                [kb, jnp.zeros((_BH_B, _S_B - Q, D), kb.dtype)], axis=1)
            rolled = pltpu.roll(kpad, o0, axis=1)
            r = jax.lax.broadcasted_iota(jnp.int32, kpad.shape, 1)
            mask = jnp.logical_xor(r >= o0, b != b0)
            out_ref[...] = cache_ref[...] + jnp.where(mask, rolled, 0.0)

        @pl.when(jnp.logical_not(hit))
        def _():
            out_ref[...] = cache_ref[...]

    return body


def kernel(k_cache, k, step):
    B, H, S, D = k_cache.shape
    Q = k.shape[-2]
    BH = B * H
    start = jnp.clip(jnp.asarray(step, jnp.int32) - Q, 0, S - Q)

    kc = k_cache.reshape(BH, S, D)
    kk = k.reshape(BH, Q, D)

    cache_spec = pl.BlockSpec(
        (_BH_B, _S_B, D), lambda i, j, s_ref: (i, j, 0))
    k_spec = pl.BlockSpec((_BH_B, Q, D), lambda i, j, s_ref: (i, 0, 0))

    grid_spec = pltpu.PrefetchScalarGridSpec(
        num_scalar_prefetch=1,
        grid=(BH // _BH_B, S // _S_B),
        in_specs=[cache_spec, k_spec],
        out_specs=cache_spec,
    )
    out = pl.pallas_call(
        _make_body(Q, D),
        grid_spec=grid_spec,
        out_shape=jax.ShapeDtypeStruct(kc.shape, kc.dtype),
        compiler_params=pltpu.CompilerParams(
            dimension_semantics=("parallel", "parallel"),
        ),
    )(start.reshape(1), kc, kk)
    return out.reshape(B, H, S, D)
